# baseline (speedup 1.0000x reference)
import functools

import jax
import jax.numpy as jnp
from jax.experimental import pallas as pl

B = 512


def _w_block(pos_c, sq_col, posT_ref, r_val, a, b):
    px_row = posT_ref[0:1, b * B:(b + 1) * B]
    py_row = posT_ref[1:2, b * B:(b + 1) * B]
    sq_row = px_row * px_row + py_row * py_row
    cross = jnp.dot(pos_c, posT_ref[0:2, b * B:(b + 1) * B],
                    preferred_element_type=jnp.float32)
    d2 = (sq_col + sq_row) - 2.0 * cross
    dist = jnp.sqrt(jnp.maximum(d2, 1e-12))
    row_ids = a * B + jax.lax.broadcasted_iota(jnp.int32, (B, B), 0)
    col_ids = b * B + jax.lax.broadcasted_iota(jnp.int32, (B, B), 1)
    mask = (dist <= r_val) & (row_ids != col_ids)
    return jnp.where(mask, 1.0 / (dist + 1e-6), 0.0)


def _deg_kernel(nb, posT_ref, pos_c_ref, x_ref, W1_ref, r_ref,
                dinv_ref, M1_ref):
    a = pl.program_id(0)
    r_val = r_ref[0:1, 0:1]
    pos_c = pos_c_ref[...]
    px_col = pos_c[:, 0:1]
    py_col = pos_c[:, 1:2]
    sq_col = px_col * px_col + py_col * py_col
    acc = jnp.zeros((B, 1), jnp.float32)
    for b in range(nb):
        w = _w_block(pos_c, sq_col, posT_ref, r_val, a, b)
        acc = acc + jnp.sum(w, axis=1, keepdims=True)
    deg = acc + 1.0
    dinv = jax.lax.rsqrt(deg)
    dinv_ref[...] = jnp.broadcast_to(dinv, (B, 8))
    xw = jnp.dot(x_ref[...], W1_ref[...], preferred_element_type=jnp.float32)
    M1_ref[...] = dinv * xw


def _agg_kernel(nb, relu_next, posT_ref, pos_c_ref, M_ref, dinv_ref, Wn_ref,
                bias_ref, r_ref, out_ref):
    a = pl.program_id(0)
    r_val = r_ref[0:1, 0:1]
    pos_c = pos_c_ref[...]
    px_col = pos_c[:, 0:1]
    py_col = pos_c[:, 1:2]
    sq_col = px_col * px_col + py_col * py_col
    y = M_ref[pl.ds(a * B, B), :]
    for b in range(nb):
        w = _w_block(pos_c, sq_col, posT_ref, r_val, a, b)
        y = y + jnp.dot(w, M_ref[b * B:(b + 1) * B, :],
                        preferred_element_type=jnp.float32)
    dinv = dinv_ref[:, 0:1]
    y = dinv * y + bias_ref[0:1, :]
    if relu_next:
        h = jax.nn.relu(y)
        out_ref[...] = dinv * jnp.dot(h, Wn_ref[...],
                                      preferred_element_type=jnp.float32)
    else:
        out_ref[...] = y


def kernel(x, pos, r, W1, b1, W2, b2):
    n, feat = x.shape
    h1 = W1.shape[1]
    h2 = W2.shape[1]
    nb = -(-n // B)
    np_ = nb * B

    pad = np_ - n
    fill = 1e6 + 1e3 * jnp.arange(pad, dtype=jnp.float32)
    pos_p = jnp.concatenate([pos, jnp.stack([fill, fill], axis=1)], axis=0)
    x_p = jnp.concatenate([x, jnp.zeros((pad, feat), x.dtype)], axis=0)
    posT = jnp.concatenate([pos_p.T, jnp.zeros((6, np_), jnp.float32)], axis=0)
    r_b = jnp.full((1, 128), r, jnp.float32)
    b1_2 = b1.reshape(1, h1)
    b2_2 = b2.reshape(1, h2)

    full = lambda shape: pl.BlockSpec(shape, lambda a: (0, 0))
    rowblk = lambda w: pl.BlockSpec((B, w), lambda a: (a, 0))

    dinv, M1 = pl.pallas_call(
        functools.partial(_deg_kernel, nb),
        grid=(nb,),
        in_specs=[full((8, np_)), rowblk(2), rowblk(feat), full((feat, h1)),
                  full((1, 128))],
        out_specs=[rowblk(8), rowblk(h1)],
        out_shape=[jax.ShapeDtypeStruct((np_, 8), jnp.float32),
                   jax.ShapeDtypeStruct((np_, h1), jnp.float32)],
    )(posT, pos_p, x_p, W1, r_b)

    M2 = pl.pallas_call(
        functools.partial(_agg_kernel, nb, True),
        grid=(nb,),
        in_specs=[full((8, np_)), rowblk(2), full((np_, h1)), rowblk(8),
                  full((h1, h2)), full((1, h1)), full((1, 128))],
        out_specs=rowblk(h2),
        out_shape=jax.ShapeDtypeStruct((np_, h2), jnp.float32),
    )(posT, pos_p, M1, dinv, W2, b1_2, r_b)

    out = pl.pallas_call(
        functools.partial(_agg_kernel, nb, False),
        grid=(nb,),
        in_specs=[full((8, np_)), rowblk(2), full((np_, h2)), rowblk(8),
                  full((h1, h2)), full((1, h2)), full((1, 128))],
        out_specs=rowblk(h2),
        out_shape=jax.ShapeDtypeStruct((np_, h2), jnp.float32),
    )(posT, pos_p, M2, dinv, W2, b2_2, r_b)

    return out[:n]


# speedup vs baseline: 1.7602x; 1.7602x over previous
import functools

import jax
import jax.numpy as jnp
from jax.experimental import pallas as pl

B = 512


def _w_block(pos_c, sq_col, geomT_ref, r2_val, a, b):
    sq_row = geomT_ref[2:3, b * B:(b + 1) * B]
    crossm2 = jnp.dot(pos_c, geomT_ref[0:2, b * B:(b + 1) * B],
                      preferred_element_type=jnp.float32)
    d2 = jnp.maximum((sq_col + sq_row) + crossm2, 1e-12)
    row_ids = a * B + jax.lax.broadcasted_iota(jnp.int32, (B, B), 0)
    col_ids = b * B + jax.lax.broadcasted_iota(jnp.int32, (B, B), 1)
    mask = (d2 <= r2_val) & (row_ids != col_ids)
    return jnp.where(mask, jax.lax.rsqrt(d2), 0.0)


def _sq_col(pos_c):
    px = pos_c[:, 0:1]
    py = pos_c[:, 1:2]
    return px * px + py * py


def _deg_kernel(nb, geomT_ref, pos_c_ref, x_ref, W1_ref, r2_ref,
                dinv_ref, M1_ref):
    a = pl.program_id(0)
    r2 = r2_ref[0:1, 0:1]
    pos_c = pos_c_ref[...]
    sq_col = _sq_col(pos_c)
    acc = jnp.zeros((B, 1), jnp.float32)
    for b in range(nb):
        w = _w_block(pos_c, sq_col, geomT_ref, r2, a, b)
        acc = acc + jnp.sum(w, axis=1, keepdims=True)
    deg = acc + 1.0
    dinv = jax.lax.rsqrt(deg)
    dinv_ref[...] = jnp.broadcast_to(dinv, (B, 8))
    xw = jnp.dot(x_ref[...], W1_ref[...], preferred_element_type=jnp.float32)
    M1_ref[...] = dinv * xw


def _agg_kernel(nb, relu_next, geomT_ref, pos_c_ref, M_ref, dinv_ref, Wn_ref,
                bias_ref, r2_ref, out_ref):
    a = pl.program_id(0)
    r2 = r2_ref[0:1, 0:1]
    pos_c = pos_c_ref[...]
    sq_col = _sq_col(pos_c)
    y = M_ref[pl.ds(a * B, B), :]
    for b in range(nb):
        w = _w_block(pos_c, sq_col, geomT_ref, r2, a, b)
        y = y + jnp.dot(w, M_ref[b * B:(b + 1) * B, :],
                        preferred_element_type=jnp.float32)
    dinv = dinv_ref[:, 0:1]
    y = dinv * y + bias_ref[0:1, :]
    if relu_next:
        h = jax.nn.relu(y)
        out_ref[...] = dinv * jnp.dot(h, Wn_ref[...],
                                      preferred_element_type=jnp.float32)
    else:
        out_ref[...] = y


def kernel(x, pos, r, W1, b1, W2, b2):
    n, feat = x.shape
    h1 = W1.shape[1]
    h2 = W2.shape[1]
    nb = -(-n // B)
    np_ = nb * B

    pad = np_ - n
    fill = 1e6 + 1e3 * jnp.arange(pad, dtype=jnp.float32)
    pos_p = jnp.concatenate([pos, jnp.stack([fill, fill], axis=1)], axis=0)
    x_p = jnp.concatenate([x, jnp.zeros((pad, feat), x.dtype)], axis=0)
    sq_p = jnp.sum(pos_p * pos_p, axis=1)
    geomT = jnp.concatenate([-2.0 * pos_p.T, sq_p[None, :],
                             jnp.zeros((5, np_), jnp.float32)], axis=0)
    r_f = jnp.asarray(r, jnp.float32)
    r2_b = jnp.full((1, 128), r_f * r_f, jnp.float32)
    b1_2 = b1.reshape(1, h1)
    b2_2 = b2.reshape(1, h2)

    full = lambda shape: pl.BlockSpec(shape, lambda a: (0, 0))
    rowblk = lambda w: pl.BlockSpec((B, w), lambda a: (a, 0))

    dinv, M1 = pl.pallas_call(
        functools.partial(_deg_kernel, nb),
        grid=(nb,),
        in_specs=[full((8, np_)), rowblk(2), rowblk(feat), full((feat, h1)),
                  full((1, 128))],
        out_specs=[rowblk(8), rowblk(h1)],
        out_shape=[jax.ShapeDtypeStruct((np_, 8), jnp.float32),
                   jax.ShapeDtypeStruct((np_, h1), jnp.float32)],
    )(geomT, pos_p, x_p, W1, r2_b)

    M2 = pl.pallas_call(
        functools.partial(_agg_kernel, nb, True),
        grid=(nb,),
        in_specs=[full((8, np_)), rowblk(2), full((np_, h1)), rowblk(8),
                  full((h1, h2)), full((1, h1)), full((1, 128))],
        out_specs=rowblk(h2),
        out_shape=jax.ShapeDtypeStruct((np_, h2), jnp.float32),
    )(geomT, pos_p, M1, dinv, W2, b1_2, r2_b)

    out = pl.pallas_call(
        functools.partial(_agg_kernel, nb, False),
        grid=(nb,),
        in_specs=[full((8, np_)), rowblk(2), full((np_, h2)), rowblk(8),
                  full((h1, h2)), full((1, h2)), full((1, 128))],
        out_specs=rowblk(h2),
        out_shape=jax.ShapeDtypeStruct((np_, h2), jnp.float32),
    )(geomT, pos_p, M2, dinv, W2, b2_2, r2_b)

    return out[:n]
